# baseline (speedup 1.0000x reference)
import functools

import jax
import jax.numpy as jnp
from jax.experimental import pallas as pl
from jax.experimental.pallas import tpu as pltpu


def _conv_stats_kernel(x_ref, w_ref, y_ref, stats_ref, *, kh, kw, h, wd, cin,
                       pad_lanes):
    hw = h * wd
    xb = x_ref[...].astype(jnp.bfloat16).reshape(cin, hw)
    zpad = jnp.zeros((cin, pad_lanes), jnp.bfloat16)
    flat = jnp.concatenate([zpad, xb, zpad], axis=1)

    colp = jax.lax.broadcasted_iota(jnp.int32, (1, hw + 2 * pad_lanes), 1)
    colp = (colp - pad_lanes) % wd
    zero_p = jnp.zeros_like(flat)
    flat_l = jnp.where(colp == wd - 1, zero_p, flat)
    flat_r = jnp.where(colp == 0, zero_p, flat)

    parts = []
    for i in range(kh):
        di = i - (kh - 1) // 2
        for j in range(kw):
            dj = j - (kw - 1) // 2
            src = flat_l if dj < 0 else (flat_r if dj > 0 else flat)
            start = pad_lanes + di * wd + dj
            parts.append(jax.lax.slice(src, (0, start), (cin, start + hw)))
    rhs = jnp.concatenate(parts, axis=0)

    acc = jnp.dot(w_ref[...], rhs,
                  preferred_element_type=jnp.float32)
    y_ref[...] = acc.astype(y_ref.dtype)
    stats_ref[:, 0:1] = jnp.sum(acc, axis=1, keepdims=True)
    stats_ref[:, 1:2] = jnp.sum(acc * acc, axis=1, keepdims=True)


def _bn_hsw_kernel(y_ref, scale_ref, shift_ref, out_ref, *, cout, h, wd):
    z = y_ref[...].astype(jnp.float32) * scale_ref[...] + shift_ref[...]
    z = z * jnp.clip(z + 3.0, 0.0, 6.0) * (1.0 / 6.0)
    out_ref[...] = z.reshape(cout, h, wd)


def kernel(x_nchw, w, gamma, beta):
    n, cin, h, wd = x_nchw.shape
    cout, cin_w, kh, kw = w.shape
    assert cin_w == cin
    hw = h * wd
    kk = kh * kw
    pad_lanes = wd + 8

    wt = jnp.transpose(w, (0, 2, 3, 1)).reshape(cout, kk * cin)
    wt = wt.astype(jnp.bfloat16)

    vmem_limit = 56 * 1024 * 1024

    y, stats = pl.pallas_call(
        functools.partial(_conv_stats_kernel, kh=kh, kw=kw, h=h, wd=wd,
                          cin=cin, pad_lanes=pad_lanes),
        out_shape=(jax.ShapeDtypeStruct((n, cout, hw), jnp.bfloat16),
                   jax.ShapeDtypeStruct((n, cout, 2), jnp.float32)),
        grid=(n,),
        in_specs=[pl.BlockSpec((None, cin, h, wd), lambda b: (b, 0, 0, 0)),
                  pl.BlockSpec((cout, kk * cin), lambda b: (0, 0))],
        out_specs=(pl.BlockSpec((None, cout, hw), lambda b: (b, 0, 0)),
                   pl.BlockSpec((None, cout, 2), lambda b: (b, 0, 0))),
        compiler_params=pltpu.CompilerParams(
            dimension_semantics=("parallel",),
            vmem_limit_bytes=vmem_limit),
        cost_estimate=pl.CostEstimate(
            flops=2 * n * hw * kk * cin * cout,
            transcendentals=0,
            bytes_accessed=(n * cin * hw * 4 + cout * kk * cin * 2
                            + n * cout * hw * 2 + n * cout * 2 * 4)),
    )(x_nchw, wt)

    m_real = float(n * hw)
    ssum = jnp.sum(stats, axis=0)
    mean = ssum[:, 0] * (1.0 / m_real)
    var = jnp.maximum(ssum[:, 1] * (1.0 / m_real) - mean * mean, 0.0)
    inv_std = jax.lax.rsqrt(var + 1e-5)
    g = gamma.astype(jnp.float32)
    scale = (g * inv_std).reshape(cout, 1)
    shift = (beta.astype(jnp.float32) - mean * g * inv_std).reshape(cout, 1)

    out = pl.pallas_call(
        functools.partial(_bn_hsw_kernel, cout=cout, h=h, wd=wd),
        out_shape=jax.ShapeDtypeStruct((n, cout, h, wd), jnp.float32),
        grid=(n,),
        in_specs=[pl.BlockSpec((None, cout, hw), lambda b: (b, 0, 0)),
                  pl.BlockSpec((cout, 1), lambda b: (0, 0)),
                  pl.BlockSpec((cout, 1), lambda b: (0, 0))],
        out_specs=pl.BlockSpec((None, cout, h, wd), lambda b: (b, 0, 0, 0)),
        compiler_params=pltpu.CompilerParams(
            dimension_semantics=("parallel",),
            vmem_limit_bytes=vmem_limit),
        cost_estimate=pl.CostEstimate(
            flops=8 * n * cout * hw,
            transcendentals=0,
            bytes_accessed=n * cout * hw * 6 + cout * 8),
    )(y, scale, shift)

    return out


# speedup vs baseline: 3.7026x; 3.7026x over previous
import functools

import jax
import jax.numpy as jnp
from jax.experimental import pallas as pl
from jax.experimental.pallas import tpu as pltpu


def _conv_stats_kernel(x_ref, w_ref, y_ref, stats_ref, *, kh, kw, h, wd, cin):
    hw = h * wd
    ph, pw = (kh - 1) // 2, (kw - 1) // 2
    xb = x_ref[...].astype(jnp.bfloat16)
    zc = jnp.zeros((h, pw, cin), jnp.bfloat16)
    xb = jnp.concatenate([zc, xb, zc], axis=1)
    zr = jnp.zeros((ph, wd + 2 * pw, cin), jnp.bfloat16)
    xb = jnp.concatenate([zr, xb, zr], axis=0)

    parts = []
    for i in range(kh):
        for j in range(kw):
            win = xb[i:i + h, j:j + wd, :]
            parts.append(win.reshape(hw, cin))
    patches = jnp.concatenate(parts, axis=1)

    acc = jnp.dot(patches, w_ref[...],
                  preferred_element_type=jnp.float32)
    y_ref[...] = acc.astype(y_ref.dtype)
    stats_ref[0:1, :] = jnp.sum(acc, axis=0, keepdims=True)
    stats_ref[1:2, :] = jnp.sum(acc * acc, axis=0, keepdims=True)


def _bn_hsw_kernel(y_ref, scale_ref, shift_ref, out_ref):
    z = y_ref[...].astype(jnp.float32) * scale_ref[...] + shift_ref[...]
    out_ref[...] = z * jnp.clip(z + 3.0, 0.0, 6.0) * (1.0 / 6.0)


def kernel(x_nchw, w, gamma, beta):
    n, cin, h, wd = x_nchw.shape
    cout, cin_w, kh, kw = w.shape
    assert cin_w == cin
    hw = h * wd
    kk = kh * kw

    x_nhwc = jnp.transpose(x_nchw, (0, 2, 3, 1))

    wt = jnp.transpose(w, (2, 3, 1, 0)).reshape(kk * cin, cout)
    wt = wt.astype(jnp.bfloat16)

    vmem_limit = 56 * 1024 * 1024

    y, stats = pl.pallas_call(
        functools.partial(_conv_stats_kernel, kh=kh, kw=kw, h=h, wd=wd,
                          cin=cin),
        out_shape=(jax.ShapeDtypeStruct((n, hw, cout), jnp.bfloat16),
                   jax.ShapeDtypeStruct((n, 2, cout), jnp.float32)),
        grid=(n,),
        in_specs=[pl.BlockSpec((None, h, wd, cin), lambda b: (b, 0, 0, 0)),
                  pl.BlockSpec((kk * cin, cout), lambda b: (0, 0))],
        out_specs=(pl.BlockSpec((None, hw, cout), lambda b: (b, 0, 0)),
                   pl.BlockSpec((None, 2, cout), lambda b: (b, 0, 0))),
        compiler_params=pltpu.CompilerParams(
            dimension_semantics=("parallel",),
            vmem_limit_bytes=vmem_limit),
        cost_estimate=pl.CostEstimate(
            flops=2 * n * hw * kk * cin * cout,
            transcendentals=0,
            bytes_accessed=(n * hw * cin * 4 + kk * cin * cout * 2
                            + n * hw * cout * 2 + n * 2 * cout * 4)),
    )(x_nhwc, wt)

    m_real = float(n * hw)
    ssum = jnp.sum(stats, axis=0)
    mean = ssum[0] * (1.0 / m_real)
    var = jnp.maximum(ssum[1] * (1.0 / m_real) - mean * mean, 0.0)
    inv_std = jax.lax.rsqrt(var + 1e-5)
    g = gamma.astype(jnp.float32)
    scale = (g * inv_std).reshape(1, cout)
    shift = (beta.astype(jnp.float32) - mean * g * inv_std).reshape(1, cout)

    out2d = pl.pallas_call(
        _bn_hsw_kernel,
        out_shape=jax.ShapeDtypeStruct((n, hw, cout), jnp.float32),
        grid=(n,),
        in_specs=[pl.BlockSpec((None, hw, cout), lambda b: (b, 0, 0)),
                  pl.BlockSpec((1, cout), lambda b: (0, 0)),
                  pl.BlockSpec((1, cout), lambda b: (0, 0))],
        out_specs=pl.BlockSpec((None, hw, cout), lambda b: (b, 0, 0)),
        compiler_params=pltpu.CompilerParams(
            dimension_semantics=("parallel",),
            vmem_limit_bytes=vmem_limit),
        cost_estimate=pl.CostEstimate(
            flops=8 * n * hw * cout,
            transcendentals=0,
            bytes_accessed=n * hw * cout * 6 + 2 * cout * 4),
    )(y, scale, shift)

    out = out2d.reshape(n, h, wd, cout)
    return jnp.transpose(out, (0, 3, 1, 2))


# speedup vs baseline: 3.9800x; 1.0749x over previous
import functools

import jax
import jax.numpy as jnp
from jax.experimental import pallas as pl
from jax.experimental.pallas import tpu as pltpu


def _conv_stats_kernel(x_ref, w_ref, y_ref, stats_ref, *, kh, kw, h, wd, cin,
                       nb):
    hw = h * wd
    ph, pw = (kh - 1) // 2, (kw - 1) // 2
    zc = jnp.zeros((h, pw, cin), jnp.bfloat16)
    zr = jnp.zeros((ph, wd + 2 * pw, cin), jnp.bfloat16)

    batch_patches = []
    for b in range(nb):
        xb = x_ref[b].astype(jnp.bfloat16)
        xb = jnp.concatenate([zc, xb, zc], axis=1)
        xb = jnp.concatenate([zr, xb, zr], axis=0)
        parts = []
        for i in range(kh):
            for j in range(kw):
                win = xb[i:i + h, j:j + wd, :]
                parts.append(win.reshape(hw, cin))
        batch_patches.append(jnp.concatenate(parts, axis=1))
    patches = jnp.concatenate(batch_patches, axis=0)

    acc = jnp.dot(patches, w_ref[...],
                  preferred_element_type=jnp.float32)
    y_ref[...] = acc.astype(y_ref.dtype).reshape(nb, hw, -1)
    stats_ref[0:1, :] = jnp.sum(acc, axis=0, keepdims=True)
    stats_ref[1:2, :] = jnp.sum(acc * acc, axis=0, keepdims=True)


def _bn_hsw_kernel(y_ref, scale_ref, shift_ref, out_ref):
    z = y_ref[...].astype(jnp.float32) * scale_ref[...] + shift_ref[...]
    out_ref[...] = z * jnp.clip(z + 3.0, 0.0, 6.0) * (1.0 / 6.0)


def kernel(x_nchw, w, gamma, beta):
    n, cin, h, wd = x_nchw.shape
    cout, cin_w, kh, kw = w.shape
    assert cin_w == cin
    hw = h * wd
    kk = kh * kw

    x_nhwc = jnp.transpose(x_nchw, (0, 2, 3, 1))

    wt = jnp.transpose(w, (2, 3, 1, 0)).reshape(kk * cin, cout)
    wt = wt.astype(jnp.bfloat16)

    vmem_limit = 56 * 1024 * 1024
    nb = 2 if n % 2 == 0 else 1
    ng = n // nb

    y, stats = pl.pallas_call(
        functools.partial(_conv_stats_kernel, kh=kh, kw=kw, h=h, wd=wd,
                          cin=cin, nb=nb),
        out_shape=(jax.ShapeDtypeStruct((n, hw, cout), jnp.bfloat16),
                   jax.ShapeDtypeStruct((ng, 2, cout), jnp.float32)),
        grid=(ng,),
        in_specs=[pl.BlockSpec((nb, h, wd, cin), lambda b: (b, 0, 0, 0)),
                  pl.BlockSpec((kk * cin, cout), lambda b: (0, 0))],
        out_specs=(pl.BlockSpec((nb, hw, cout), lambda b: (b, 0, 0)),
                   pl.BlockSpec((None, 2, cout), lambda b: (b, 0, 0))),
        compiler_params=pltpu.CompilerParams(
            dimension_semantics=("parallel",),
            vmem_limit_bytes=vmem_limit),
        cost_estimate=pl.CostEstimate(
            flops=2 * n * hw * kk * cin * cout,
            transcendentals=0,
            bytes_accessed=(n * hw * cin * 4 + kk * cin * cout * 2
                            + n * hw * cout * 2 + n * 2 * cout * 4)),
    )(x_nhwc, wt)

    m_real = float(n * hw)
    ssum = jnp.sum(stats, axis=0)
    mean = ssum[0] * (1.0 / m_real)
    var = jnp.maximum(ssum[1] * (1.0 / m_real) - mean * mean, 0.0)
    inv_std = jax.lax.rsqrt(var + 1e-5)
    g = gamma.astype(jnp.float32)
    scale = (g * inv_std).reshape(1, cout)
    shift = (beta.astype(jnp.float32) - mean * g * inv_std).reshape(1, cout)

    out2d = pl.pallas_call(
        _bn_hsw_kernel,
        out_shape=jax.ShapeDtypeStruct((n, hw, cout), jnp.float32),
        grid=(ng,),
        in_specs=[pl.BlockSpec((nb, hw, cout), lambda b: (b, 0, 0)),
                  pl.BlockSpec((1, cout), lambda b: (0, 0)),
                  pl.BlockSpec((1, cout), lambda b: (0, 0))],
        out_specs=pl.BlockSpec((nb, hw, cout), lambda b: (b, 0, 0)),
        compiler_params=pltpu.CompilerParams(
            dimension_semantics=("parallel",),
            vmem_limit_bytes=vmem_limit),
        cost_estimate=pl.CostEstimate(
            flops=8 * n * hw * cout,
            transcendentals=0,
            bytes_accessed=n * hw * cout * 6 + 2 * cout * 4),
    )(y, scale, shift)

    out = out2d.reshape(n, h, wd, cout)
    return jnp.transpose(out, (0, 3, 1, 2))


# speedup vs baseline: 4.0335x; 1.0134x over previous
import functools

import jax
import jax.numpy as jnp
from jax.experimental import pallas as pl
from jax.experimental.pallas import tpu as pltpu


def _conv_stats_kernel(x_ref, w_ref, y_ref, stats_ref, *, kh, kw, h, wd, cin,
                       nb):
    hw = h * wd
    ph, pw = (kh - 1) // 2, (kw - 1) // 2
    zc = jnp.zeros((h, pw, cin), jnp.bfloat16)
    zr = jnp.zeros((ph, wd + 2 * pw, cin), jnp.bfloat16)

    batch_patches = []
    for b in range(nb):
        xb = x_ref[b].astype(jnp.bfloat16)
        xb = jnp.concatenate([zc, xb, zc], axis=1)
        xb = jnp.concatenate([zr, xb, zr], axis=0)
        cols = [xb[:, j:j + wd, :] for j in range(kw)]
        parts = []
        for i in range(kh):
            for j in range(kw):
                parts.append(cols[j][i:i + h].reshape(hw, cin))
        batch_patches.append(jnp.concatenate(parts, axis=1))
    patches = jnp.concatenate(batch_patches, axis=0)

    acc = jnp.dot(patches, w_ref[...],
                  preferred_element_type=jnp.float32)
    y_ref[...] = acc.astype(y_ref.dtype).reshape(nb, hw, -1)
    stats_ref[0:1, :] = jnp.sum(acc, axis=0, keepdims=True)
    stats_ref[1:2, :] = jnp.sum(acc * acc, axis=0, keepdims=True)


def _bn_hsw_kernel(y_ref, stats_ref, gamma_ref, beta_ref, out_ref, *, inv_m):
    s = jnp.sum(stats_ref[...], axis=0)
    mean = s[0:1] * inv_m
    var = jnp.maximum(s[1:2] * inv_m - mean * mean, 0.0)
    inv_std = jax.lax.rsqrt(var + 1e-5)
    scale = gamma_ref[...] * inv_std
    shift = beta_ref[...] - mean * scale
    z = y_ref[...].astype(jnp.float32) * scale + shift
    out_ref[...] = z * jnp.clip(z + 3.0, 0.0, 6.0) * (1.0 / 6.0)


def kernel(x_nchw, w, gamma, beta):
    n, cin, h, wd = x_nchw.shape
    cout, cin_w, kh, kw = w.shape
    assert cin_w == cin
    hw = h * wd
    kk = kh * kw

    x_nhwc = jnp.transpose(x_nchw, (0, 2, 3, 1))

    wt = jnp.transpose(w, (2, 3, 1, 0)).reshape(kk * cin, cout)
    wt = wt.astype(jnp.bfloat16)

    vmem_limit = 56 * 1024 * 1024
    nb = 2 if n % 2 == 0 else 1
    ng = n // nb

    y, stats = pl.pallas_call(
        functools.partial(_conv_stats_kernel, kh=kh, kw=kw, h=h, wd=wd,
                          cin=cin, nb=nb),
        out_shape=(jax.ShapeDtypeStruct((n, hw, cout), jnp.bfloat16),
                   jax.ShapeDtypeStruct((ng, 2, cout), jnp.float32)),
        grid=(ng,),
        in_specs=[pl.BlockSpec((nb, h, wd, cin), lambda b: (b, 0, 0, 0)),
                  pl.BlockSpec((kk * cin, cout), lambda b: (0, 0))],
        out_specs=(pl.BlockSpec((nb, hw, cout), lambda b: (b, 0, 0)),
                   pl.BlockSpec((None, 2, cout), lambda b: (b, 0, 0))),
        compiler_params=pltpu.CompilerParams(
            dimension_semantics=("parallel",),
            vmem_limit_bytes=vmem_limit),
        cost_estimate=pl.CostEstimate(
            flops=2 * n * hw * kk * cin * cout,
            transcendentals=0,
            bytes_accessed=(n * hw * cin * 4 + kk * cin * cout * 2
                            + n * hw * cout * 2 + n * 2 * cout * 4)),
    )(x_nhwc, wt)

    gamma2 = gamma.astype(jnp.float32).reshape(1, cout)
    beta2 = beta.astype(jnp.float32).reshape(1, cout)

    out2d = pl.pallas_call(
        functools.partial(_bn_hsw_kernel, inv_m=1.0 / float(n * hw)),
        out_shape=jax.ShapeDtypeStruct((n, hw, cout), jnp.float32),
        grid=(ng,),
        in_specs=[pl.BlockSpec((nb, hw, cout), lambda b: (b, 0, 0)),
                  pl.BlockSpec((ng, 2, cout), lambda b: (0, 0, 0)),
                  pl.BlockSpec((1, cout), lambda b: (0, 0)),
                  pl.BlockSpec((1, cout), lambda b: (0, 0))],
        out_specs=pl.BlockSpec((nb, hw, cout), lambda b: (b, 0, 0)),
        compiler_params=pltpu.CompilerParams(
            dimension_semantics=("parallel",),
            vmem_limit_bytes=vmem_limit),
        cost_estimate=pl.CostEstimate(
            flops=8 * n * hw * cout,
            transcendentals=0,
            bytes_accessed=n * hw * cout * 6 + ng * 2 * cout * 4
                           + 2 * cout * 4),
    )(y, stats, gamma2, beta2)

    out = out2d.reshape(n, h, wd, cout)
    return jnp.transpose(out, (0, 3, 1, 2))


# speedup vs baseline: 4.1044x; 1.0176x over previous
import functools

import jax
import jax.numpy as jnp
from jax.experimental import pallas as pl
from jax.experimental.pallas import tpu as pltpu


def _conv_stats_kernel(x_ref, w_ref, y_ref, stats_ref, *, kh, kw, h, wd, cin,
                       nb):
    hw = h * wd
    ph, pw = (kh - 1) // 2, (kw - 1) // 2
    zc = jnp.zeros((h, pw, cin), jnp.bfloat16)
    zr = jnp.zeros((ph, wd + 2 * pw, cin), jnp.bfloat16)

    batch_patches = []
    for b in range(nb):
        xb = x_ref[b].astype(jnp.bfloat16)
        xb = jnp.concatenate([zc, xb, zc], axis=1)
        xb = jnp.concatenate([zr, xb, zr], axis=0)
        cols = [xb[:, j:j + wd, :] for j in range(kw)]
        parts = []
        for i in range(kh):
            for j in range(kw):
                parts.append(cols[j][i:i + h].reshape(hw, cin))
        batch_patches.append(jnp.concatenate(parts, axis=1))
    patches = jnp.concatenate(batch_patches, axis=0)

    acc = jnp.dot(patches, w_ref[...],
                  preferred_element_type=jnp.float32)
    y_ref[...] = acc.astype(y_ref.dtype).reshape(nb, hw, -1)
    stats_ref[0:1, :] = jnp.sum(acc, axis=0, keepdims=True)
    stats_ref[1:2, :] = jnp.sum(acc * acc, axis=0, keepdims=True)


def _bn_hsw_kernel(y_ref, stats_ref, gamma_ref, beta_ref, out_ref, *, inv_m):
    s = jnp.sum(stats_ref[...], axis=0)
    mean = s[0:1] * inv_m
    var = jnp.maximum(s[1:2] * inv_m - mean * mean, 0.0)
    inv_std = jax.lax.rsqrt(var + 1e-5)
    scale = gamma_ref[...] * inv_std
    shift = beta_ref[...] - mean * scale
    z = y_ref[...].astype(jnp.float32) * scale + shift
    out_ref[...] = z * jnp.clip(z + 3.0, 0.0, 6.0) * (1.0 / 6.0)


def kernel(x_nchw, w, gamma, beta):
    n, cin, h, wd = x_nchw.shape
    cout, cin_w, kh, kw = w.shape
    assert cin_w == cin
    hw = h * wd
    kk = kh * kw

    x_nhwc = jnp.transpose(x_nchw, (0, 2, 3, 1))

    wt = jnp.transpose(w, (2, 3, 1, 0)).reshape(kk * cin, cout)
    wt = wt.astype(jnp.bfloat16)

    vmem_limit = 56 * 1024 * 1024
    nb = 2 if n % 2 == 0 else 1
    ng = n // nb

    y, stats = pl.pallas_call(
        functools.partial(_conv_stats_kernel, kh=kh, kw=kw, h=h, wd=wd,
                          cin=cin, nb=nb),
        out_shape=(jax.ShapeDtypeStruct((n, hw, cout), jnp.bfloat16),
                   jax.ShapeDtypeStruct((ng, 2, cout), jnp.float32)),
        grid=(ng,),
        in_specs=[pl.BlockSpec((nb, h, wd, cin), lambda b: (b, 0, 0, 0)),
                  pl.BlockSpec((kk * cin, cout), lambda b: (0, 0))],
        out_specs=(pl.BlockSpec((nb, hw, cout), lambda b: (b, 0, 0)),
                   pl.BlockSpec((None, 2, cout), lambda b: (b, 0, 0))),
        compiler_params=pltpu.CompilerParams(
            dimension_semantics=("parallel",),
            vmem_limit_bytes=vmem_limit),
        cost_estimate=pl.CostEstimate(
            flops=2 * n * hw * kk * cin * cout,
            transcendentals=0,
            bytes_accessed=(n * hw * cin * 4 + kk * cin * cout * 2
                            + n * hw * cout * 2 + n * 2 * cout * 4)),
    )(x_nhwc, wt)

    gamma2 = gamma.astype(jnp.float32).reshape(1, cout)
    beta2 = beta.astype(jnp.float32).reshape(1, cout)

    nb2 = 4 if n % 4 == 0 else nb
    ng2 = n // nb2

    out2d = pl.pallas_call(
        functools.partial(_bn_hsw_kernel, inv_m=1.0 / float(n * hw)),
        out_shape=jax.ShapeDtypeStruct((n, hw, cout), jnp.float32),
        grid=(ng2,),
        in_specs=[pl.BlockSpec((nb2, hw, cout), lambda b: (b, 0, 0)),
                  pl.BlockSpec((ng, 2, cout), lambda b: (0, 0, 0)),
                  pl.BlockSpec((1, cout), lambda b: (0, 0)),
                  pl.BlockSpec((1, cout), lambda b: (0, 0))],
        out_specs=pl.BlockSpec((nb2, hw, cout), lambda b: (b, 0, 0)),
        compiler_params=pltpu.CompilerParams(
            dimension_semantics=("parallel",),
            vmem_limit_bytes=vmem_limit),
        cost_estimate=pl.CostEstimate(
            flops=8 * n * hw * cout,
            transcendentals=0,
            bytes_accessed=n * hw * cout * 6 + ng * 2 * cout * 4
                           + 2 * cout * 4),
    )(y, stats, gamma2, beta2)

    out = out2d.reshape(n, h, wd, cout)
    return jnp.transpose(out, (0, 3, 1, 2))
